# baseline (speedup 1.0000x reference)
import functools

import jax
import jax.numpy as jnp
from jax import lax
from jax.experimental import pallas as pl
from jax.experimental.pallas import tpu as pltpu
from jax.experimental.pallas import tpu_sc as plsc

VOCAB = 1000000
ACT_NUM = 1000
EMB = 64
B = 16384
SL = 8
SPLIT = 524288
NSLABS_LO = SPLIT // SL

NC = 2
NS = 16
L = 16
NW = NC * NS
BPW = B // NW
NQ = EMB // L
G = BPW // L

_mesh = plsc.VectorSubcoreMesh(core_axis_name="c", subcore_axis_name="s")


@functools.partial(
    pl.kernel,
    out_type=jax.ShapeDtypeStruct((B,), jnp.float32),
    mesh=_mesh,
    scratch_types=[
        pltpu.VMEM((BPW,), jnp.int32),
        pltpu.VMEM((BPW,), jnp.int32),
        pltpu.VMEM((BPW,), jnp.int32),
        [pltpu.VMEM((L, SL, EMB), jnp.float32)] * 2,
        [pltpu.VMEM((L, SL, EMB), jnp.float32)] * 2,
        [pltpu.VMEM((L, SL, EMB), jnp.float32)] * 2,
        pltpu.VMEM((EMB,), jnp.float32),
        pltpu.VMEM((BPW,), jnp.float32),
        pltpu.VMEM((L, L), jnp.float32),
        [pltpu.SemaphoreType.DMA] * 2,
    ],
    compiler_params=pltpu.CompilerParams(needs_layout_passes=False),
)
def _transe_sc(head_hbm, tail_hbm, act_hbm, ctlo_hbm, cthi_hbm, at_hbm,
               c_hbm, out_hbm,
               hidx_v, tidx_v, aidx_v, h_v, t_v, a_v, c_v, out_v,
               pbuf_v, sems):
    wid = lax.axis_index("s") * NC + lax.axis_index("c")
    base = pl.multiple_of(wid * BPW, BPW)

    pltpu.sync_copy(head_hbm.at[pl.ds(base, BPW)], hidx_v)
    pltpu.sync_copy(tail_hbm.at[pl.ds(base, BPW)], tidx_v)
    pltpu.sync_copy(act_hbm.at[pl.ds(base, BPW)], aidx_v)
    pltpu.sync_copy(c_hbm, c_v)

    cs = [c_v[pl.ds(q * L, L)] for q in range(NQ)]
    lane = jnp.arange(L, dtype=jnp.int32)
    inv = jnp.float32(1.0 / EMB)
    seven = jnp.int32(7)
    one = jnp.int32(1)

    def fire_split(slab, dst, sem):
        cond = slab < jnp.int32(NSLABS_LO)

        @pl.when(cond)
        def _():
            pltpu.async_copy(ctlo_hbm.at[slab], dst, sem)

        @pl.when(jnp.logical_not(cond))
        def _():
            pltpu.async_copy(cthi_hbm.at[slab - jnp.int32(NSLABS_LO)],
                             dst, sem)

    def fire(g, buf):
        rsl = pl.ds(pl.multiple_of(g * L, L), L)
        hslab = lax.shift_right_logical(hidx_v[rsl], 3)
        tslab = lax.shift_right_logical(tidx_v[rsl], 3)
        aslab = lax.shift_right_logical(aidx_v[rsl], 3)
        for i in range(L):
            fire_split(hslab[i], h_v[buf].at[i], sems[buf])
            fire_split(tslab[i], t_v[buf].at[i], sems[buf])
            pltpu.async_copy(
                at_hbm.at[aslab[i]], a_v[buf].at[i], sems[buf])

    def drain(buf):
        for i in range(L):
            pltpu.make_async_copy(
                ctlo_hbm.at[0], h_v[buf].at[i], sems[buf]).wait()
            pltpu.make_async_copy(
                ctlo_hbm.at[0], t_v[buf].at[i], sems[buf]).wait()
            pltpu.make_async_copy(
                at_hbm.at[0], a_v[buf].at[i], sems[buf]).wait()

    def compute(g, buf):
        row0 = pl.multiple_of(g * L, L)
        rsl = pl.ds(row0, L)
        phv = lax.bitwise_and(hidx_v[rsl], seven)
        ptv = lax.bitwise_and(tidx_v[rsl], seven)
        pav = lax.bitwise_and(aidx_v[rsl], seven)
        for i in range(L):
            ph = phv[i]
            pt = ptv[i]
            pa = pav[i]
            d = None
            for q in range(NQ):
                sl2 = pl.ds(q * L, L)
                hq = h_v[buf][i, ph, sl2]
                tq = t_v[buf][i, pt, sl2]
                aq = a_v[buf][i, pa, sl2]
                dq = jnp.abs(hq + aq - tq + cs[q])
                d = dq if d is None else d + dq
            plsc.store_scatter(
                pbuf_v, [lane, jnp.full((L,), i, jnp.int32)], d)
        acc = pbuf_v[0, :]
        for rr in range(1, L):
            acc = acc + pbuf_v[rr, :]
        out_v[rsl] = acc * inv

    fire(0, 0)

    def body(k, carry):
        g0 = lax.mul(k, jnp.int32(2))
        fire(g0 + 1, 1)
        drain(0)
        compute(g0, 0)
        fire(jnp.minimum(g0 + 2, jnp.int32(G - 1)), 0)
        drain(1)
        compute(g0 + 1, 1)
        return carry

    lax.fori_loop(0, G // 2, body, 0)
    drain(0)

    pltpu.sync_copy(out_v, out_hbm.at[pl.ds(base, BPW)])


def kernel(data, concept_table, act_table, begin, end):
    head = data[:, 0].astype(jnp.int32)
    act = data[:, 1].astype(jnp.int32)
    tail = data[:, 2].astype(jnp.int32)
    cvec = (begin - end).reshape(EMB).astype(jnp.float32)
    ctlo = concept_table[:SPLIT].reshape(NSLABS_LO, SL, EMB)
    cthi = concept_table[SPLIT:].reshape((VOCAB - SPLIT) // SL, SL, EMB)
    at3 = act_table.reshape(ACT_NUM // SL, SL, EMB)
    return _transe_sc(head, tail, act, ctlo, cthi, at3, cvec)


# speedup vs baseline: 1.3875x; 1.3875x over previous
import functools

import jax
import jax.numpy as jnp
from jax import lax
from jax.experimental import pallas as pl
from jax.experimental.pallas import tpu as pltpu
from jax.experimental.pallas import tpu_sc as plsc

VOCAB = 1000000
ACT_NUM = 1000
EMB = 64
B = 16384
SL = 8

NC = 2
NS = 16
L = 16
NW = NC * NS
BPW = B // NW
NQ = EMB // L
G = BPW // L

_mesh = plsc.VectorSubcoreMesh(core_axis_name="c", subcore_axis_name="s")


@functools.partial(
    pl.kernel,
    out_type=jax.ShapeDtypeStruct((B,), jnp.float32),
    mesh=_mesh,
    scratch_types=[
        pltpu.VMEM((BPW,), jnp.int32),
        pltpu.VMEM((BPW,), jnp.int32),
        pltpu.VMEM((BPW,), jnp.int32),
        [pltpu.VMEM((L, SL, EMB), jnp.float32)] * 2,
        [pltpu.VMEM((L, SL, EMB), jnp.float32)] * 2,
        [pltpu.VMEM((L, SL, EMB), jnp.float32)] * 2,
        pltpu.VMEM((EMB,), jnp.float32),
        pltpu.VMEM((BPW,), jnp.float32),
        pltpu.VMEM((L, L), jnp.float32),
        [pltpu.SemaphoreType.DMA] * 2,
    ],
    compiler_params=pltpu.CompilerParams(needs_layout_passes=False),
)
def _transe_sc(head_hbm, tail_hbm, act_hbm, ct_hbm, at_hbm,
               c_hbm, out_hbm,
               hidx_v, tidx_v, aidx_v, h_v, t_v, a_v, c_v, out_v,
               pbuf_v, sems):
    wid = lax.axis_index("s") * NC + lax.axis_index("c")
    base = pl.multiple_of(wid * BPW, BPW)

    pltpu.sync_copy(head_hbm.at[pl.ds(base, BPW)], hidx_v)
    pltpu.sync_copy(tail_hbm.at[pl.ds(base, BPW)], tidx_v)
    pltpu.sync_copy(act_hbm.at[pl.ds(base, BPW)], aidx_v)
    pltpu.sync_copy(c_hbm, c_v)

    cs = [c_v[pl.ds(q * L, L)] for q in range(NQ)]
    lane = jnp.arange(L, dtype=jnp.int32)
    inv = jnp.float32(1.0 / EMB)
    seven = jnp.int32(7)
    one = jnp.int32(1)

    def fire(g, buf):
        rsl = pl.ds(pl.multiple_of(g * L, L), L)
        hslab = lax.shift_right_logical(hidx_v[rsl], 3)
        tslab = lax.shift_right_logical(tidx_v[rsl], 3)
        aslab = lax.shift_right_logical(aidx_v[rsl], 3)
        for i in range(L):
            pltpu.async_copy(ct_hbm.at[hslab[i]], h_v[buf].at[i], sems[buf])
            pltpu.async_copy(ct_hbm.at[tslab[i]], t_v[buf].at[i], sems[buf])
            pltpu.async_copy(at_hbm.at[aslab[i]], a_v[buf].at[i], sems[buf])

    def drain(buf):
        for i in range(L):
            pltpu.make_async_copy(
                ct_hbm.at[0], h_v[buf].at[i], sems[buf]).wait()
            pltpu.make_async_copy(
                ct_hbm.at[0], t_v[buf].at[i], sems[buf]).wait()
            pltpu.make_async_copy(
                at_hbm.at[0], a_v[buf].at[i], sems[buf]).wait()

    def compute(g, buf):
        row0 = pl.multiple_of(g * L, L)
        rsl = pl.ds(row0, L)
        phv = lax.bitwise_and(hidx_v[rsl], seven)
        ptv = lax.bitwise_and(tidx_v[rsl], seven)
        pav = lax.bitwise_and(aidx_v[rsl], seven)
        for i in range(L):
            ph = phv[i]
            pt = ptv[i]
            pa = pav[i]
            d = None
            for q in range(NQ):
                sl2 = pl.ds(q * L, L)
                hq = h_v[buf][i, ph, sl2]
                tq = t_v[buf][i, pt, sl2]
                aq = a_v[buf][i, pa, sl2]
                dq = jnp.abs(hq + aq - tq + cs[q])
                d = dq if d is None else d + dq
            plsc.store_scatter(
                pbuf_v, [lane, jnp.full((L,), i, jnp.int32)], d)
        acc = pbuf_v[0, :]
        for rr in range(1, L):
            acc = acc + pbuf_v[rr, :]
        out_v[rsl] = acc * inv

    fire(0, 0)

    def body(k, carry):
        g0 = lax.mul(k, jnp.int32(2))
        fire(g0 + 1, 1)
        drain(0)
        compute(g0, 0)
        fire(jnp.minimum(g0 + 2, jnp.int32(G - 1)), 0)
        drain(1)
        compute(g0 + 1, 1)
        return carry

    lax.fori_loop(0, G // 2, body, 0)
    drain(0)

    pltpu.sync_copy(out_v, out_hbm.at[pl.ds(base, BPW)])


def kernel(data, concept_table, act_table, begin, end):
    head = data[:, 0].astype(jnp.int32)
    act = data[:, 1].astype(jnp.int32)
    tail = data[:, 2].astype(jnp.int32)
    cvec = (begin - end).reshape(EMB).astype(jnp.float32)
    ct3 = concept_table.reshape(VOCAB // SL, SL, EMB)
    at3 = act_table.reshape(ACT_NUM // SL, SL, EMB)
    return _transe_sc(head, tail, act, ct3, at3, cvec)


# speedup vs baseline: 1.3940x; 1.0047x over previous
import functools

import jax
import jax.numpy as jnp
from jax import lax
from jax.experimental import pallas as pl
from jax.experimental.pallas import tpu as pltpu
from jax.experimental.pallas import tpu_sc as plsc

VOCAB = 1000000
ACT_NUM = 1000
EMB = 64
B = 16384
SL = 8

NC = 2
NS = 16
L = 16
NW = NC * NS
BPW = B // NW
NQ = EMB // L
G = BPW // L

_mesh = plsc.VectorSubcoreMesh(core_axis_name="c", subcore_axis_name="s")


@functools.partial(
    pl.kernel,
    out_type=jax.ShapeDtypeStruct((B,), jnp.float32),
    mesh=_mesh,
    scratch_types=[
        pltpu.VMEM((BPW,), jnp.int32),
        pltpu.VMEM((BPW,), jnp.int32),
        pltpu.VMEM((BPW,), jnp.int32),
        [pltpu.VMEM((L, SL, EMB), jnp.float32)] * 2,
        [pltpu.VMEM((L, SL, EMB), jnp.float32)] * 2,
        [pltpu.VMEM((L, SL, EMB), jnp.float32)] * 2,
        pltpu.VMEM((EMB,), jnp.float32),
        pltpu.VMEM((BPW,), jnp.float32),
        pltpu.VMEM((L, L), jnp.float32),
        [pltpu.SemaphoreType.DMA] * 2,
    ],
    compiler_params=pltpu.CompilerParams(needs_layout_passes=False),
)
def _transe_sc(head_hbm, tail_hbm, act_hbm, ct_hbm, at_hbm,
               c_hbm, out_hbm,
               hidx_v, tidx_v, aidx_v, h_v, t_v, a_v, c_v, out_v,
               pbuf_v, sems):
    wid = lax.axis_index("s") * NC + lax.axis_index("c")
    base = pl.multiple_of(wid * BPW, BPW)

    pltpu.sync_copy(head_hbm.at[pl.ds(base, BPW)], hidx_v)
    pltpu.sync_copy(tail_hbm.at[pl.ds(base, BPW)], tidx_v)
    pltpu.sync_copy(act_hbm.at[pl.ds(base, BPW)], aidx_v)
    pltpu.sync_copy(c_hbm, c_v)

    cs = [c_v[pl.ds(q * L, L)] for q in range(NQ)]
    lane = jnp.arange(L, dtype=jnp.int32)
    inv = jnp.float32(1.0 / EMB)
    seven = jnp.int32(7)
    one = jnp.int32(1)

    def fire(g, buf):
        rsl = pl.ds(pl.multiple_of(g * L, L), L)
        hslab = lax.shift_right_logical(hidx_v[rsl], 3)
        tslab = lax.shift_right_logical(tidx_v[rsl], 3)
        aslab = lax.shift_right_logical(aidx_v[rsl], 3)
        for i in range(L):
            pltpu.async_copy(ct_hbm.at[hslab[i]], h_v[buf].at[i], sems[buf])
            pltpu.async_copy(ct_hbm.at[tslab[i]], t_v[buf].at[i], sems[buf])
            pltpu.async_copy(at_hbm.at[aslab[i]], a_v[buf].at[i], sems[buf])

    def drain(buf):
        pltpu.make_async_copy(
            ct_hbm.at[pl.ds(0, L)], h_v[buf], sems[buf]).wait()
        pltpu.make_async_copy(
            ct_hbm.at[pl.ds(0, L)], t_v[buf], sems[buf]).wait()
        pltpu.make_async_copy(
            at_hbm.at[pl.ds(0, L)], a_v[buf], sems[buf]).wait()

    def compute(g, buf):
        row0 = pl.multiple_of(g * L, L)
        rsl = pl.ds(row0, L)
        phv = lax.bitwise_and(hidx_v[rsl], seven)
        ptv = lax.bitwise_and(tidx_v[rsl], seven)
        pav = lax.bitwise_and(aidx_v[rsl], seven)
        for i in range(L):
            ph = phv[i]
            pt = ptv[i]
            pa = pav[i]
            d = None
            for q in range(NQ):
                sl2 = pl.ds(q * L, L)
                hq = h_v[buf][i, ph, sl2]
                tq = t_v[buf][i, pt, sl2]
                aq = a_v[buf][i, pa, sl2]
                dq = jnp.abs(hq + aq - tq + cs[q])
                d = dq if d is None else d + dq
            plsc.store_scatter(
                pbuf_v, [lane, jnp.full((L,), i, jnp.int32)], d)
        acc = pbuf_v[0, :]
        for rr in range(1, L):
            acc = acc + pbuf_v[rr, :]
        out_v[rsl] = acc * inv

    fire(0, 0)

    def body(k, carry):
        g0 = lax.mul(k, jnp.int32(2))
        fire(g0 + 1, 1)
        drain(0)
        compute(g0, 0)
        fire(jnp.minimum(g0 + 2, jnp.int32(G - 1)), 0)
        drain(1)
        compute(g0 + 1, 1)
        return carry

    lax.fori_loop(0, G // 2, body, 0)
    drain(0)

    pltpu.sync_copy(out_v, out_hbm.at[pl.ds(base, BPW)])


def kernel(data, concept_table, act_table, begin, end):
    head = data[:, 0].astype(jnp.int32)
    act = data[:, 1].astype(jnp.int32)
    tail = data[:, 2].astype(jnp.int32)
    cvec = (begin - end).reshape(EMB).astype(jnp.float32)
    ct3 = concept_table.reshape(VOCAB // SL, SL, EMB)
    at3 = act_table.reshape(ACT_NUM // SL, SL, EMB)
    return _transe_sc(head, tail, act, ct3, at3, cvec)


# speedup vs baseline: 1.4975x; 1.0743x over previous
import functools

import jax
import jax.numpy as jnp
from jax import lax
from jax.experimental import pallas as pl
from jax.experimental.pallas import tpu as pltpu
from jax.experimental.pallas import tpu_sc as plsc

VOCAB = 1000000
ACT_NUM = 1000
EMB = 64
B = 16384
SL = 8
W = 2 * EMB

NC = 2
NS = 16
L = 16
NW = NC * NS
BPW = B // NW
NQ = EMB // L
G = BPW // L
CHA = 256
NH = BPW // CHA
GPH = CHA // L

_mesh = plsc.VectorSubcoreMesh(core_axis_name="c", subcore_axis_name="s")


@functools.partial(
    pl.kernel,
    out_type=jax.ShapeDtypeStruct((B,), jnp.float32),
    mesh=_mesh,
    scratch_types=[
        pltpu.VMEM((BPW,), jnp.int32),
        pltpu.VMEM((BPW,), jnp.int32),
        pltpu.VMEM((BPW,), jnp.int32),
        [pltpu.VMEM((L, SL, EMB), jnp.float32)] * 2,
        [pltpu.VMEM((L, SL, EMB), jnp.float32)] * 2,
        [pltpu.VMEM((CHA,), jnp.int32)] * NH,
        pltpu.VMEM((CHA, W), jnp.float32),
        pltpu.VMEM((EMB,), jnp.float32),
        pltpu.VMEM((BPW,), jnp.float32),
        pltpu.VMEM((L, L), jnp.float32),
        [pltpu.SemaphoreType.DMA] * 2,
        pltpu.SemaphoreType.DMA,
    ],
    compiler_params=pltpu.CompilerParams(needs_layout_passes=False),
)
def _transe_sc(head_hbm, tail_hbm, act_hbm, ct_hbm, at_hbm,
               c_hbm, out_hbm,
               hidx_v, tidx_v, aidx_v, h_v, t_v, amaj_v, a_v, c_v, out_v,
               pbuf_v, sems, asem):
    wid = lax.axis_index("s") * NC + lax.axis_index("c")
    base = pl.multiple_of(wid * BPW, BPW)

    pltpu.sync_copy(head_hbm.at[pl.ds(base, BPW)], hidx_v)
    pltpu.sync_copy(tail_hbm.at[pl.ds(base, BPW)], tidx_v)
    pltpu.sync_copy(act_hbm.at[pl.ds(base, BPW)], aidx_v)
    pltpu.sync_copy(c_hbm, c_v)

    for k in range(BPW // L):
        hh, off = k // GPH, (k % GPH) * L
        amaj_v[hh][pl.ds(off, L)] = lax.shift_right_logical(
            aidx_v[pl.ds(k * L, L)], 1)

    cs = [c_v[pl.ds(q * L, L)] for q in range(NQ)]
    lane = jnp.arange(L, dtype=jnp.int32)
    inv = jnp.float32(1.0 / EMB)
    seven = jnp.int32(7)
    one = jnp.int32(1)

    def fire(g, buf):
        rsl = pl.ds(pl.multiple_of(g * L, L), L)
        hslab = lax.shift_right_logical(hidx_v[rsl], 3)
        tslab = lax.shift_right_logical(tidx_v[rsl], 3)
        for i in range(L):
            pltpu.async_copy(ct_hbm.at[hslab[i]], h_v[buf].at[i], sems[buf])
            pltpu.async_copy(ct_hbm.at[tslab[i]], t_v[buf].at[i], sems[buf])

    def drain(buf):
        pltpu.make_async_copy(
            ct_hbm.at[pl.ds(0, L)], h_v[buf], sems[buf]).wait()
        pltpu.make_async_copy(
            ct_hbm.at[pl.ds(0, L)], t_v[buf], sems[buf]).wait()

    def compute(g, buf, hh):
        row0 = pl.multiple_of(g * L, L)
        rsl = pl.ds(row0, L)
        arow0 = row0 - hh * CHA
        phv = lax.bitwise_and(hidx_v[rsl], seven)
        ptv = lax.bitwise_and(tidx_v[rsl], seven)
        pav = lax.shift_left(lax.bitwise_and(aidx_v[rsl], one), 6)
        for i in range(L):
            ph = phv[i]
            pt = ptv[i]
            pa = pav[i]
            d = None
            for q in range(NQ):
                sl2 = pl.ds(q * L, L)
                hq = h_v[buf][i, ph, sl2]
                tq = t_v[buf][i, pt, sl2]
                aq = a_v[arow0 + i, pl.ds(pl.multiple_of(pa + q * L, L), L)]
                dq = jnp.abs(hq + aq - tq + cs[q])
                d = dq if d is None else d + dq
            plsc.store_scatter(
                pbuf_v, [lane, jnp.full((L,), i, jnp.int32)], d)
        acc = pbuf_v[0, :]
        for rr in range(1, L):
            acc = acc + pbuf_v[rr, :]
        out_v[rsl] = acc * inv

    fire(0, 0)

    for hh in range(NH):
        pltpu.async_copy(at_hbm.at[amaj_v[hh]], a_v, asem).wait()

        def body(k, carry, hh=hh):
            g0 = lax.add(lax.mul(k, jnp.int32(2)), jnp.int32(hh * GPH))
            fire(g0 + 1, 1)
            drain(0)
            compute(g0, 0, hh)
            fire(jnp.minimum(g0 + 2, jnp.int32(G - 1)), 0)
            drain(1)
            compute(g0 + 1, 1, hh)
            return carry

        lax.fori_loop(0, GPH // 2, body, 0)
    drain(0)

    pltpu.sync_copy(out_v, out_hbm.at[pl.ds(base, BPW)])


def kernel(data, concept_table, act_table, begin, end):
    head = data[:, 0].astype(jnp.int32)
    act = data[:, 1].astype(jnp.int32)
    tail = data[:, 2].astype(jnp.int32)
    cvec = (begin - end).reshape(EMB).astype(jnp.float32)
    ct3 = concept_table.reshape(VOCAB // SL, SL, EMB)
    at2 = act_table.reshape(ACT_NUM // 2, W)
    return _transe_sc(head, tail, act, ct3, at2, cvec)
